# baseline (speedup 1.0000x reference)
import functools

import jax
import jax.numpy as jnp
from jax import lax
from jax.experimental import pallas as pl
from jax.experimental.pallas import tpu as pltpu
from jax.experimental.pallas import tpu_sc as plsc

N_FIELDS = 26
VOCAB = 100000
DIM = 64
BATCH = 16384

VC = 2048
NVC = (VOCAB + VC - 1) // VC
PROWS = NVC * (VC // 2)

NC = 2
NS = 16
LANES = 16
NW = NC * NS
BW = BATCH // NW
GB = 128
G = BW // GB
NBUF = 2


def _pack_body(t2_ref, o_ref):
    blk = t2_ref[0]
    t2 = blk.T
    o_ref[0, :, 0:DIM] = t2[0:VC // 2, :]
    o_ref[0, :, DIM:2 * DIM] = t2[VC // 2:VC, :]


def _pack_tables(t2):
    return pl.pallas_call(
        _pack_body,
        grid=(N_FIELDS, NVC),
        in_specs=[pl.BlockSpec((1, DIM, VC), lambda f, vc: (f, 0, vc))],
        out_specs=pl.BlockSpec((1, VC // 2, 2 * DIM), lambda f, vc: (f, vc, 0)),
        out_shape=jax.ShapeDtypeStruct((N_FIELDS, PROWS, 2 * DIM), jnp.float32),
    )(t2)


def _feature_sum_call():
    mesh = plsc.VectorSubcoreMesh(core_axis_name="c", subcore_axis_name="s")

    @functools.partial(
        pl.kernel,
        mesh=mesh,
        out_type=jax.ShapeDtypeStruct((BATCH, DIM), jnp.float32),
        compiler_params=pltpu.CompilerParams(needs_layout_passes=False),
        scratch_types=[
            pltpu.VMEM((BW * N_FIELDS,), jnp.int32),
            pltpu.VMEM((N_FIELDS, BW), jnp.int32),
            pltpu.VMEM((N_FIELDS, BW), jnp.int32),
            pltpu.VMEM((GB, 2 * DIM), jnp.float32),
            pltpu.VMEM((GB, 2 * DIM), jnp.float32),
            pltpu.VMEM((DIM, BW), jnp.float32),
            pltpu.VMEM((GB, DIM), jnp.float32),
            pltpu.SemaphoreType.DMA,
            pltpu.SemaphoreType.DMA,
        ],
    )
    def k(xw_hbm, tab_hbm, out_hbm, xv, idx, hoff, b0, b1, acc, obuf, s0, s1):
        bufs = (b0, b1)
        sems = (s0, s1)
        wid = lax.axis_index("s") * NC + lax.axis_index("c")
        base = wid * BW

        pltpu.sync_copy(xw_hbm.at[pl.ds(base * N_FIELDS, BW * N_FIELDS)], xv)

        lane = jnp.arange(LANES, dtype=jnp.int32)
        gpos = lane * N_FIELDS

        for f in range(N_FIELDS):
            def build(t, _, f=f):
                v = plsc.load_gather(xv, [gpos + (t * LANES * N_FIELDS + f)])
                p = ((v >> 11) << 10) | (v & 1023)
                idx[f, pl.ds(t * LANES, LANES)] = p + f * PROWS
                hoff[f, pl.ds(t * LANES, LANES)] = ((v >> 10) & 1) * DIM
                return 0
            lax.fori_loop(0, BW // LANES, build, 0)

        def issue(s):
            g, f = divmod(s, N_FIELDS)
            return pltpu.async_copy(
                tab_hbm.at[idx.at[f, pl.ds(g * GB, GB)]],
                bufs[s % NBUF], sems[s % NBUF])

        def accum(s):
            buf = bufs[s % NBUF]
            g, f = divmod(s, N_FIELDS)

            def grp(r0, _, g=g, f=f, buf=buf, first=(f == 0)):
                rows = lane + r0 * LANES
                off = g * GB + r0 * LANES
                cols0 = hoff[f, pl.ds(off, LANES)]
                if first:
                    def bd(dd, _):
                        acc[dd, pl.ds(off, LANES)] = plsc.load_gather(
                            buf, [rows, cols0 + dd])
                        return 0
                else:
                    def bd(dd, _):
                        plsc.addupdate(acc.at[dd, pl.ds(off, LANES)],
                                       plsc.load_gather(buf, [rows, cols0 + dd]))
                        return 0
                lax.fori_loop(0, DIM, bd, 0)
                return 0

            lax.fori_loop(0, GB // LANES, grp, 0)

        total = G * N_FIELDS
        pending = {}
        for s in range(min(NBUF - 1, total)):
            pending[s] = issue(s)
        for s in range(total):
            nxt = s + NBUF - 1
            if nxt < total:
                pending[nxt] = issue(nxt)
            pending.pop(s).wait()
            accum(s)

        for g in range(G):
            def tr(r, _, g=g):
                b = g * GB + r
                def trc(c, _, b=b, r=r):
                    obuf[r, pl.ds(c * LANES, LANES)] = plsc.load_gather(
                        acc, [lane + c * LANES, jnp.broadcast_to(b, (LANES,))])
                    return 0
                lax.fori_loop(0, DIM // LANES, trc, 0)
                return 0
            lax.fori_loop(0, GB, tr, 0)
            pltpu.sync_copy(obuf, out_hbm.at[pl.ds(base + g * GB, GB), :])

    return k


def kernel(x, tables):
    t2 = jnp.transpose(tables, (0, 2, 1))
    packed = _pack_tables(t2).reshape(N_FIELDS * PROWS, 2 * DIM)
    xflat = x.reshape(BATCH * N_FIELDS)
    return _feature_sum_call()(xflat, packed)


# speedup vs baseline: 1.1505x; 1.1505x over previous
import functools

import jax
import jax.numpy as jnp
from jax import lax
from jax.experimental import pallas as pl
from jax.experimental.pallas import tpu as pltpu
from jax.experimental.pallas import tpu_sc as plsc

N_FIELDS = 26
VOCAB = 100000
DIM = 64
BATCH = 16384

VC = 2048
NVC = (VOCAB + VC - 1) // VC
PROWS = NVC * (VC // 2)

NC = 2
NS = 16
LANES = 16
NW = NC * NS
BW = BATCH // NW
GB = 128
G = BW // GB
NBUF = 2


def _pack_body(t2_ref, o_ref):
    blk = t2_ref[0]
    r = lax.broadcasted_iota(jnp.int32, (DIM, DIM), 0)
    c = lax.broadcasted_iota(jnp.int32, (DIM, DIM), 1)
    eye = (r == c).astype(jnp.float32)
    t2 = lax.dot_general(blk, eye, (((0,), (0,)), ((), ())),
                         preferred_element_type=jnp.float32)
    o_ref[0, :, 0:DIM] = t2[0:VC // 2, :]
    o_ref[0, :, DIM:2 * DIM] = t2[VC // 2:VC, :]


def _pack_tables(t2):
    return pl.pallas_call(
        _pack_body,
        grid=(N_FIELDS, NVC),
        in_specs=[pl.BlockSpec((1, DIM, VC), lambda f, vc: (f, 0, vc))],
        out_specs=pl.BlockSpec((1, VC // 2, 2 * DIM), lambda f, vc: (f, vc, 0)),
        out_shape=jax.ShapeDtypeStruct((N_FIELDS, PROWS, 2 * DIM), jnp.float32),
    )(t2)


def _feature_sum_call():
    mesh = plsc.VectorSubcoreMesh(core_axis_name="c", subcore_axis_name="s")

    @functools.partial(
        pl.kernel,
        mesh=mesh,
        out_type=jax.ShapeDtypeStruct((BATCH, DIM), jnp.float32),
        compiler_params=pltpu.CompilerParams(needs_layout_passes=False),
        scratch_types=[
            pltpu.VMEM((BW * N_FIELDS,), jnp.int32),
            pltpu.VMEM((N_FIELDS, BW), jnp.int32),
            pltpu.VMEM((N_FIELDS, BW + LANES), jnp.int32),
            pltpu.VMEM((GB, 2 * DIM), jnp.float32),
            pltpu.VMEM((GB, 2 * DIM), jnp.float32),
            pltpu.VMEM((GB, DIM), jnp.float32),
            pltpu.SemaphoreType.DMA,
            pltpu.SemaphoreType.DMA,
        ],
    )
    def k(xw_hbm, tab_hbm, out_hbm, xv, idx, hoff, b0, b1, acc, s0, s1):
        bufs = (b0, b1)
        sems = (s0, s1)
        wid = lax.axis_index("s") * NC + lax.axis_index("c")
        base = wid * BW

        pltpu.sync_copy(xw_hbm.at[pl.ds(base * N_FIELDS, BW * N_FIELDS)], xv)

        lane = jnp.arange(LANES, dtype=jnp.int32)
        gpos = lane * N_FIELDS

        for f in range(N_FIELDS):
            def build(t, _, f=f):
                v = plsc.load_gather(xv, [gpos + (t * LANES * N_FIELDS + f)])
                p = ((v >> 11) << 10) | (v & 1023)
                idx[f, pl.ds(t * LANES, LANES)] = p + f * PROWS
                hoff[f, pl.ds(t * LANES, LANES)] = ((v >> 10) & 1) * DIM
                return 0
            lax.fori_loop(0, BW // LANES, build, 0)

        def issue(s):
            g, f = divmod(s, N_FIELDS)
            return pltpu.async_copy(
                tab_hbm.at[idx.at[f, pl.ds(g * GB, GB)]],
                bufs[s % NBUF], sems[s % NBUF])

        def accum(s):
            buf = bufs[s % NBUF]
            g, f = divmod(s, N_FIELDS)

            def row(r, _, g=g, f=f, buf=buf, first=(f == 0)):
                h = hoff[f, pl.ds(g * GB + r, LANES)][0]
                if first:
                    for c in range(DIM // LANES):
                        acc[r, pl.ds(c * LANES, LANES)] = (
                            buf[r, pl.ds(h + c * LANES, LANES)])
                else:
                    for c in range(DIM // LANES):
                        plsc.addupdate(acc.at[r, pl.ds(c * LANES, LANES)],
                                       buf[r, pl.ds(h + c * LANES, LANES)])
                return 0

            lax.fori_loop(0, GB, row, 0)

        total = G * N_FIELDS
        pending = {}
        for s in range(min(NBUF - 1, total)):
            pending[s] = issue(s)
        for s in range(total):
            nxt = s + NBUF - 1
            if nxt < total:
                pending[nxt] = issue(nxt)
            pending.pop(s).wait()
            accum(s)
            if s % N_FIELDS == N_FIELDS - 1:
                g = s // N_FIELDS
                pltpu.sync_copy(acc, out_hbm.at[pl.ds(base + g * GB, GB), :])

    return k


def kernel(x, tables):
    t2 = jnp.transpose(tables, (0, 2, 1))
    packed = _pack_tables(t2).reshape(N_FIELDS * PROWS, 2 * DIM)
    xflat = x.reshape(BATCH * N_FIELDS)
    return _feature_sum_call()(xflat, packed)


# speedup vs baseline: 1.1946x; 1.0384x over previous
import functools

import jax
import jax.numpy as jnp
from jax import lax
from jax.experimental import pallas as pl
from jax.experimental.pallas import tpu as pltpu
from jax.experimental.pallas import tpu_sc as plsc

N_FIELDS = 26
VOCAB = 100000
DIM = 64
BATCH = 16384

NC = 2
NS = 16
LANES = 16
NW = NC * NS
BW = BATCH // NW
GB = 128
G = BW // GB
NBUF = 3


def _feature_sum_call():
    mesh = plsc.VectorSubcoreMesh(core_axis_name="c", subcore_axis_name="s")

    @functools.partial(
        pl.kernel,
        mesh=mesh,
        out_type=jax.ShapeDtypeStruct((BATCH, DIM), jnp.float32),
        compiler_params=pltpu.CompilerParams(
            use_tc_tiling_on_sc=False, needs_layout_passes=False),
        scratch_types=[
            pltpu.VMEM((BW * N_FIELDS,), jnp.int32),
            pltpu.VMEM((N_FIELDS, BW), jnp.int32),
            pltpu.VMEM((GB, DIM), jnp.float32),
            pltpu.VMEM((GB, DIM), jnp.float32),
            pltpu.VMEM((GB, DIM), jnp.float32),
            pltpu.VMEM((GB, DIM), jnp.float32),
            pltpu.SemaphoreType.DMA,
            pltpu.SemaphoreType.DMA,
            pltpu.SemaphoreType.DMA,
        ],
    )
    def k(xw_hbm, tab_hbm, out_hbm, xv, idx, b0, b1, b2, acc, s0, s1, s2):
        bufs = (b0, b1, b2)
        sems = (s0, s1, s2)
        wid = lax.axis_index("s") * NC + lax.axis_index("c")
        base = wid * BW

        pltpu.sync_copy(xw_hbm.at[pl.ds(base * N_FIELDS, BW * N_FIELDS)], xv)

        lane = jnp.arange(LANES, dtype=jnp.int32) * N_FIELDS
        for f in range(N_FIELDS):
            def build(t, _, f=f):
                pos = lane + (t * LANES * N_FIELDS + f)
                v = plsc.load_gather(xv, [pos]) + f * VOCAB
                idx[f, pl.ds(t * LANES, LANES)] = v
                return 0
            lax.fori_loop(0, BW // LANES, build, 0)

        def issue(s):
            g, f = divmod(s, N_FIELDS)
            return pltpu.async_copy(
                tab_hbm.at[idx.at[f, pl.ds(g * GB, GB)]],
                bufs[s % NBUF], sems[s % NBUF])

        def accum(s):
            buf = bufs[s % NBUF]
            f = s % N_FIELDS
            if f == 0:
                def bd(r, _):
                    for c in range(DIM // LANES):
                        sl = pl.ds(c * LANES, LANES)
                        acc[r, sl] = buf[r, sl]
                    return 0
            else:
                def bd(r, _):
                    for c in range(DIM // LANES):
                        sl = pl.ds(c * LANES, LANES)
                        plsc.addupdate(acc.at[r, sl], buf[r, sl])
                    return 0
            lax.fori_loop(0, GB, bd, 0)

        total = G * N_FIELDS
        pending = {}
        for s in range(min(NBUF - 1, total)):
            pending[s] = issue(s)
        for s in range(total):
            nxt = s + NBUF - 1
            if nxt < total:
                pending[nxt] = issue(nxt)
            pending.pop(s).wait()
            accum(s)
            if s % N_FIELDS == N_FIELDS - 1:
                g = s // N_FIELDS
                pltpu.sync_copy(acc, out_hbm.at[pl.ds(base + g * GB, GB), :])

    return k


def kernel(x, tables):
    xflat = x.reshape(BATCH * N_FIELDS)
    tab = tables.reshape(N_FIELDS * VOCAB, DIM)
    return _feature_sum_call()(xflat, tab)
